# baseline (speedup 1.0000x reference)
import functools

import jax
import jax.numpy as jnp
import numpy as np
from jax import lax
from jax.experimental import pallas as pl
from jax.experimental.pallas import tpu as pltpu
from jax.experimental.pallas import tpu_sc as plsc

_info = plsc.get_sparse_core_info()
_NC, _NS, _L = _info.num_cores, _info.num_subcores, _info.num_lanes
_NW = _NC * _NS
_N, _K = 16384, 4
_B = _N * _K
_D = 10
_V = 100
_RPW = _N // _NW
_BPW = _RPW * _K
_OPW = _BPW * _D
_PERIOD = 5
_NMACRO = _OPW // (_PERIOD * _L)

_mesh = plsc.VectorSubcoreMesh(core_axis_name="c", subcore_axis_name="s")

_T = np.arange(_PERIOD * _L, dtype=np.int32).reshape(_PERIOD, _L)
_PAT = np.concatenate([
    (_T // _D) // _K,
    (_T // _D) % _K,
    _T % _D,
    _T // _D,
    _T // (_K * _D),
    _T % (_K * _D),
]).astype(np.int32)


@functools.partial(
    pl.kernel,
    mesh=_mesh,
    compiler_params=pltpu.CompilerParams(
        needs_layout_passes=False, use_tc_tiling_on_sc=False),
    out_type=jax.ShapeDtypeStruct((_N, _K * _D), jnp.float32),
    scratch_types=[
        pltpu.VMEM((_RPW, _K), jnp.int32),
        pltpu.VMEM((_V, _D), jnp.float32),
        pltpu.VMEM((_RPW, _K * _D), jnp.float32),
        pltpu.VMEM((6 * _PERIOD, _L), jnp.int32),
    ],
)
def _gather_kernel(x_hbm, table_hbm, pat_hbm, out_hbm, idx_v, table_v,
                   out_v, pat_v):
    wid = lax.axis_index("s") * _NC + lax.axis_index("c")
    row0 = wid * _RPW
    pltpu.sync_copy(x_hbm.at[pl.ds(row0, _RPW)], idx_v)
    pltpu.sync_copy(table_hbm, table_v)
    pltpu.sync_copy(pat_hbm, pat_v)

    xr = [pat_v[g] for g in range(_PERIOD)]
    xc = [pat_v[_PERIOD + g] for g in range(_PERIOD)]
    tc = [pat_v[2 * _PERIOD + g] for g in range(_PERIOD)]
    oa = [pat_v[4 * _PERIOD + g] for g in range(_PERIOD)]
    oc = [pat_v[5 * _PERIOD + g] for g in range(_PERIOD)]

    @plsc.parallel_loop(0, _NMACRO, unroll=4)
    def _(m):
        m2 = 2 * m
        for g in range(_PERIOD):
            rows = plsc.load_gather(idx_v, [xr[g] + m2, xc[g]])
            vals = plsc.load_gather(table_v, [rows, tc[g]])
            plsc.store_scatter(out_v, [oa[g] + m2, oc[g]], vals)

    pltpu.sync_copy(out_v, out_hbm.at[pl.ds(row0, _RPW)])


def kernel(x, table):
    out = _gather_kernel(x, table, jnp.asarray(_PAT))
    return out.reshape(-1, 2, 2, 10)


# speedup vs baseline: 1.0543x; 1.0543x over previous
import functools

import jax
import jax.numpy as jnp
import numpy as np
from jax import lax
from jax.experimental import pallas as pl
from jax.experimental.pallas import tpu as pltpu
from jax.experimental.pallas import tpu_sc as plsc

_info = plsc.get_sparse_core_info()
_NC, _NS, _L = _info.num_cores, _info.num_subcores, _info.num_lanes
_NW = _NC * _NS
_N, _K = 16384, 4
_B = _N * _K
_D = 10
_V = 100
_RPW = _N // _NW
_BPW = _RPW * _K
_OPW = _BPW * _D
_PERIOD = 5
_NMACRO = _OPW // (_PERIOD * _L)

_mesh = plsc.VectorSubcoreMesh(core_axis_name="c", subcore_axis_name="s")

_T = np.arange(_PERIOD * _L, dtype=np.int32).reshape(_PERIOD, _L)
_PAT = np.concatenate([
    _T // _D,
    _T % _D,
    _T // (_K * _D),
    _T % (_K * _D),
    np.zeros((1, _L), np.int32),
]).astype(np.int32)


@functools.partial(
    pl.kernel,
    mesh=_mesh,
    compiler_params=pltpu.CompilerParams(
        needs_layout_passes=False, use_tc_tiling_on_sc=False),
    out_type=jax.ShapeDtypeStruct((_N, _K * _D), jnp.float32),
    scratch_types=[
        pltpu.VMEM((_BPW // 128, 128), jnp.int32),
        pltpu.VMEM((_V, _D), jnp.float32),
        pltpu.VMEM((_RPW, _K * _D), jnp.float32),
        pltpu.VMEM((4 * _PERIOD + 1, _L), jnp.int32),
    ],
)
def _gather_kernel(x_hbm, table_hbm, pat_hbm, out_hbm, idx_v, table_v,
                   out_v, pat_v):
    wid = lax.axis_index("s") * _NC + lax.axis_index("c")
    row0 = wid * _RPW
    xrow0 = wid * (_BPW // 128)
    pltpu.sync_copy(x_hbm.at[pl.ds(xrow0, _BPW // 128)], idx_v)
    pltpu.sync_copy(table_hbm, table_v)
    pltpu.sync_copy(pat_hbm, pat_v)

    q = [pat_v[g] for g in range(_PERIOD)]
    tc = [pat_v[_PERIOD + g] for g in range(_PERIOD)]
    oa = [pat_v[2 * _PERIOD + g] for g in range(_PERIOD)]
    oc = [pat_v[3 * _PERIOD + g] for g in range(_PERIOD)]
    zz = pat_v[4 * _PERIOD]

    @plsc.parallel_loop(0, _NMACRO, unroll=8)
    def _(m):
        m2 = 2 * m
        i0 = zz + (m >> 4)
        sh = (m & 15) << 3
        for g in range(_PERIOD):
            rows = plsc.load_gather(idx_v, [i0, q[g] + sh])
            vals = plsc.load_gather(table_v, [rows, tc[g]])
            plsc.store_scatter(out_v, [oa[g] + m2, oc[g]], vals)

    pltpu.sync_copy(out_v, out_hbm.at[pl.ds(row0, _RPW)])


def kernel(x, table):
    out = _gather_kernel(x.reshape(_B // 128, 128), table, jnp.asarray(_PAT))
    return out.reshape(-1, 2, 2, 10)
